# baseline (speedup 1.0000x reference)
import functools

import jax
import jax.numpy as jnp
from jax import lax
from jax.experimental import pallas as pl
from jax.experimental.pallas import tpu as pltpu
from jax.experimental.pallas import tpu_sc as plsc

_CH = 128
_NSLAB = 32
_RB = 2000


def _cdiv(a, b):
    return (a + b - 1) // b


def _fill_const(ref, rows, d, val):
    vec = jnp.full((16,), val, jnp.float32)

    def row(i, carry):
        for jj in range(d // 16):
            ref[i, pl.ds(jj * 16, 16)] = vec
        return carry

    lax.fori_loop(0, rows, row, 0)


def _sc_degree(dst3, npad):
    nslab, C, ch = dst3.shape
    rpt = npad // 16
    mesh = plsc.VectorSubcoreMesh(core_axis_name="c", subcore_axis_name="s")

    def body(dst_hbm, out_hbm, didx, obuf, zbuf, acc):
        c = lax.axis_index("c")
        s = lax.axis_index("s")
        slab = c * 16 + s
        pltpu.sync_copy(dst_hbm.at[slab], didx)
        _fill_const(obuf, ch, 16, 1.0)
        _fill_const(zbuf, ch, 16, 0.0)
        for t in range(rpt // ch):
            pltpu.sync_copy(zbuf, acc.at[pl.ds(s * rpt + t * ch, ch)])
        plsc.subcore_barrier()

        def step(j, carry):
            pltpu.sync_copy(obuf, acc.at[didx.at[j]], add=True)
            return carry

        lax.fori_loop(0, C, step, 0)
        plsc.subcore_barrier()
        pltpu.sync_copy(acc.at[pl.ds(s * rpt, rpt)],
                        out_hbm.at[c, pl.ds(s * rpt, rpt)])

    f = pl.kernel(
        body,
        out_type=jax.ShapeDtypeStruct((2, npad, 16), jnp.float32),
        mesh=mesh,
        compiler_params=pltpu.CompilerParams(use_tc_tiling_on_sc=False),
        scratch_types=[
            pltpu.VMEM((C, ch), jnp.int32),
            pltpu.VMEM((ch, 16), jnp.float32),
            pltpu.VMEM((ch, 16), jnp.float32),
            pltpu.VMEM_SHARED((npad, 16), jnp.float32),
        ],
    )
    return f(dst3)


def _sc_agg(u, src3, dst3, npad):
    n, d = u.shape
    nslab, C, ch = src3.shape
    rpt = npad // 16
    mesh = plsc.VectorSubcoreMesh(core_axis_name="c", subcore_axis_name="s")

    def body(u_hbm, src_hbm, dst_hbm, out_hbm, sidx2, didx2, rows, acc,
             isem, gsem, ssem):
        c = lax.axis_index("c")
        s = lax.axis_index("s")
        slab = c * 16 + s
        _fill_const(rows.at[0], ch, d, 0.0)
        for t in range(rpt // ch):
            pltpu.sync_copy(rows.at[0], acc.at[pl.ds(s * rpt + t * ch, ch)])
        plsc.subcore_barrier()

        pltpu.sync_copy(src_hbm.at[slab, 0], sidx2.at[0])
        pltpu.sync_copy(dst_hbm.at[slab, 0], didx2.at[0])
        pltpu.async_copy(u_hbm.at[sidx2.at[0]], rows.at[0], gsem)

        def step(j, carry):
            b = j % 2
            pltpu.make_async_copy(u_hbm.at[sidx2.at[b]], rows.at[b],
                                  gsem).wait()

            @pl.when(j > 0)
            def _():
                pltpu.make_async_copy(rows.at[1 - b],
                                      acc.at[didx2.at[1 - b]], ssem).wait()

            pltpu.async_copy(rows.at[b], acc.at[didx2.at[b]], ssem, add=True)

            @pl.when(j + 1 < C)
            def _():
                pltpu.sync_copy(src_hbm.at[slab, j + 1], sidx2.at[1 - b])
                pltpu.sync_copy(dst_hbm.at[slab, j + 1], didx2.at[1 - b])
                pltpu.async_copy(u_hbm.at[sidx2.at[1 - b]], rows.at[1 - b],
                                 gsem)

            return carry

        lax.fori_loop(0, C, step, 0)
        bl = (C - 1) % 2
        pltpu.make_async_copy(rows.at[bl], acc.at[didx2.at[bl]], ssem).wait()
        plsc.subcore_barrier()
        pltpu.sync_copy(acc.at[pl.ds(s * rpt, rpt)],
                        out_hbm.at[c, pl.ds(s * rpt, rpt)])

    f = pl.kernel(
        body,
        out_type=jax.ShapeDtypeStruct((2, npad, d), jnp.float32),
        mesh=mesh,
        compiler_params=pltpu.CompilerParams(use_tc_tiling_on_sc=False),
        scratch_types=[
            pltpu.VMEM((2, ch), jnp.int32),
            pltpu.VMEM((2, ch), jnp.int32),
            pltpu.VMEM((2, ch, d), jnp.float32),
            pltpu.VMEM_SHARED((npad, d), jnp.float32),
            pltpu.SemaphoreType.DMA,
            pltpu.SemaphoreType.DMA,
            pltpu.SemaphoreType.DMA,
        ],
    )
    return f(u, src3, dst3)


def _mm_body(x_ref, w_ref, o_ref):
    o_ref[...] = jnp.dot(x_ref[...], w_ref[...],
                         preferred_element_type=jnp.float32)


def _tc_matmul(x, w):
    n, k = x.shape
    m = w.shape[1]
    return pl.pallas_call(
        _mm_body,
        grid=(n // _RB,),
        in_specs=[
            pl.BlockSpec((_RB, k), lambda i: (i, 0)),
            pl.BlockSpec((k, m), lambda i: (0, 0)),
        ],
        out_specs=pl.BlockSpec((_RB, m), lambda i: (i, 0)),
        out_shape=jax.ShapeDtypeStruct((n, m), jnp.float32),
    )(x, w)


def _scale_body(h_ref, a_ref, b_ref, o_ref):
    dinv = lax.rsqrt(1.0 + a_ref[:, 0:1] + b_ref[:, 0:1])
    o_ref[...] = h_ref[...] * dinv


def _tc_scale(h, d0, d1):
    n, m = h.shape
    return pl.pallas_call(
        _scale_body,
        grid=(n // _RB,),
        in_specs=[
            pl.BlockSpec((_RB, m), lambda i: (i, 0)),
            pl.BlockSpec((_RB, 16), lambda i: (i, 0)),
            pl.BlockSpec((_RB, 16), lambda i: (i, 0)),
        ],
        out_specs=pl.BlockSpec((_RB, m), lambda i: (i, 0)),
        out_shape=jax.ShapeDtypeStruct((n, m), jnp.float32),
    )(h, d0, d1)


def _l2_body(p0_ref, p1_ref, u1_ref, a_ref, b_ref, b1_ref, w2_ref, o_ref):
    dinv = lax.rsqrt(1.0 + a_ref[:, 0:1] + b_ref[:, 0:1])
    t = dinv * (p0_ref[...] + p1_ref[...] + u1_ref[...]) + b1_ref[...]
    t = jnp.maximum(t, 0.0)
    o_ref[...] = dinv * jnp.dot(t, w2_ref[...],
                                preferred_element_type=jnp.float32)


def _tc_layer2(p0, p1, u1, d0, d1, b1, w2):
    n, m = u1.shape
    ncls = w2.shape[1]
    return pl.pallas_call(
        _l2_body,
        grid=(n // _RB,),
        in_specs=[
            pl.BlockSpec((_RB, m), lambda i: (i, 0)),
            pl.BlockSpec((_RB, m), lambda i: (i, 0)),
            pl.BlockSpec((_RB, m), lambda i: (i, 0)),
            pl.BlockSpec((_RB, 16), lambda i: (i, 0)),
            pl.BlockSpec((_RB, 16), lambda i: (i, 0)),
            pl.BlockSpec((1, m), lambda i: (0, 0)),
            pl.BlockSpec((m, ncls), lambda i: (0, 0)),
        ],
        out_specs=pl.BlockSpec((_RB, ncls), lambda i: (i, 0)),
        out_shape=jax.ShapeDtypeStruct((n, ncls), jnp.float32),
    )(p0, p1, u1, d0, d1, b1, w2)


def _fin_body(q0_ref, q1_ref, u2_ref, a_ref, b_ref, b2_ref, o_ref):
    dinv = lax.rsqrt(1.0 + a_ref[:, 0:1] + b_ref[:, 0:1])
    z = dinv * (q0_ref[...] + q1_ref[...] + u2_ref[...]) + b2_ref[...]
    z = z - jnp.max(z, axis=1, keepdims=True)
    e = jnp.exp(z)
    o_ref[...] = e / jnp.sum(e, axis=1, keepdims=True)


def _tc_final(q0, q1, u2, d0, d1, b2):
    n, ncls = u2.shape
    return pl.pallas_call(
        _fin_body,
        grid=(n // _RB,),
        in_specs=[
            pl.BlockSpec((_RB, ncls), lambda i: (i, 0)),
            pl.BlockSpec((_RB, ncls), lambda i: (i, 0)),
            pl.BlockSpec((_RB, ncls), lambda i: (i, 0)),
            pl.BlockSpec((_RB, 16), lambda i: (i, 0)),
            pl.BlockSpec((_RB, 16), lambda i: (i, 0)),
            pl.BlockSpec((1, ncls), lambda i: (0, 0)),
        ],
        out_specs=pl.BlockSpec((_RB, ncls), lambda i: (i, 0)),
        out_shape=jax.ShapeDtypeStruct((n, ncls), jnp.float32),
    )(q0, q1, u2, d0, d1, b2)


def kernel(x, edge_index, W1, b1, W2, b2):
    n, _ = x.shape
    e = edge_index.shape[1]

    C = _cdiv(e, _NSLAB * _CH)
    epad = _NSLAB * C * _CH
    npad = (n // (16 * _CH) + 1) * (16 * _CH)

    pad = epad - e
    src = edge_index[0]
    dst = edge_index[1]
    src3 = jnp.concatenate(
        [src, jnp.zeros((pad,), jnp.int32)]).reshape(_NSLAB, C, _CH)
    dst3 = jnp.concatenate(
        [dst, jnp.full((pad,), n, jnp.int32)]).reshape(_NSLAB, C, _CH)

    deg = _sc_degree(dst3, npad)
    d0 = deg[0, :n]
    d1 = deg[1, :n]

    h1 = _tc_matmul(x, W1)
    u1 = _tc_scale(h1, d0, d1)

    p = _sc_agg(u1, src3, dst3, npad)
    u2 = _tc_layer2(p[0, :n], p[1, :n], u1, d0, d1,
                    b1.reshape(1, -1), W2)

    q = _sc_agg(u2, src3, dst3, npad)
    return _tc_final(q[0, :n], q[1, :n], u2, d0, d1, b2.reshape(1, -1))


# speedup vs baseline: 1.2319x; 1.2319x over previous
import functools

import jax
import jax.numpy as jnp
from jax import lax
from jax.experimental import pallas as pl
from jax.experimental.pallas import tpu as pltpu
from jax.experimental.pallas import tpu_sc as plsc

_CH = 128
_NSLAB = 32
_RB = 2000


def _cdiv(a, b):
    return (a + b - 1) // b


def _fill_const(ref, rows, d, val):
    vec = jnp.full((16,), val, jnp.float32)

    def row(i, carry):
        for jj in range(d // 16):
            ref[i, pl.ds(jj * 16, 16)] = vec
        return carry

    lax.fori_loop(0, rows, row, 0)


def _sc_degree(dst3, npad):
    nslab, C, ch = dst3.shape
    rpt = npad // 16
    mesh = plsc.VectorSubcoreMesh(core_axis_name="c", subcore_axis_name="s")

    def body(dst_hbm, out_hbm, didx, obuf, zbuf, acc):
        c = lax.axis_index("c")
        s = lax.axis_index("s")
        slab = c * 16 + s
        pltpu.sync_copy(dst_hbm.at[slab], didx)
        _fill_const(obuf, ch, 16, 1.0)
        _fill_const(zbuf, ch, 16, 0.0)
        for t in range(rpt // ch):
            pltpu.sync_copy(zbuf, acc.at[pl.ds(s * rpt + t * ch, ch)])
        plsc.subcore_barrier()

        def step(j, carry):
            pltpu.sync_copy(obuf, acc.at[didx.at[j]], add=True)
            return carry

        lax.fori_loop(0, C, step, 0)
        plsc.subcore_barrier()
        pltpu.sync_copy(acc.at[pl.ds(s * rpt, rpt)],
                        out_hbm.at[c, pl.ds(s * rpt, rpt)])

    f = pl.kernel(
        body,
        out_type=jax.ShapeDtypeStruct((2, npad, 16), jnp.float32),
        mesh=mesh,
        compiler_params=pltpu.CompilerParams(use_tc_tiling_on_sc=False),
        scratch_types=[
            pltpu.VMEM((C, ch), jnp.int32),
            pltpu.VMEM((ch, 16), jnp.float32),
            pltpu.VMEM((ch, 16), jnp.float32),
            pltpu.VMEM_SHARED((npad, 16), jnp.float32),
        ],
    )
    return f(dst3)


def _sc_agg(u, src3, dst3, npad):
    n, d = u.shape
    nslab, C, ch = src3.shape
    rpt = npad // 16
    mesh = plsc.VectorSubcoreMesh(core_axis_name="c", subcore_axis_name="s")

    def body(u_hbm, src_hbm, dst_hbm, out_hbm, sidx3, didx3, rows, acc,
             isem, gsem, ssem):
        c = lax.axis_index("c")
        s = lax.axis_index("s")
        slab = c * 16 + s
        _fill_const(rows.at[0], ch, d, 0.0)
        for t in range(rpt // ch):
            pltpu.sync_copy(rows.at[0], acc.at[pl.ds(s * rpt + t * ch, ch)])
        plsc.subcore_barrier()

        def load_idx(j, slot, sem):
            pltpu.async_copy(src_hbm.at[slab, j], sidx3.at[slot], sem)
            pltpu.async_copy(dst_hbm.at[slab, j], didx3.at[slot], sem)

        def wait_idx(j, slot, sem):
            pltpu.make_async_copy(src_hbm.at[slab, j], sidx3.at[slot],
                                  sem).wait()
            pltpu.make_async_copy(dst_hbm.at[slab, j], didx3.at[slot],
                                  sem).wait()

        load_idx(0, 0, isem)
        wait_idx(0, 0, isem)
        pltpu.async_copy(u_hbm.at[sidx3.at[0]], rows.at[0], gsem)
        load_idx(1, 1, isem)

        def step(j, carry):
            b2 = j % 2
            b3 = j % 3
            pltpu.make_async_copy(u_hbm.at[sidx3.at[b3]], rows.at[b2],
                                  gsem).wait()

            @pl.when(j > 0)
            def _():
                pltpu.make_async_copy(rows.at[1 - b2],
                                      acc.at[didx3.at[(j + 2) % 3]],
                                      ssem).wait()

            pltpu.async_copy(rows.at[b2], acc.at[didx3.at[b3]], ssem,
                             add=True)

            @pl.when(j + 1 < C)
            def _():
                wait_idx(j + 1, (j + 1) % 3, isem)
                pltpu.async_copy(u_hbm.at[sidx3.at[(j + 1) % 3]],
                                 rows.at[1 - b2], gsem)

            @pl.when(j + 2 < C)
            def _():
                load_idx(j + 2, (j + 2) % 3, isem)

            return carry

        lax.fori_loop(0, C, step, 0)
        bl2 = (C - 1) % 2
        bl3 = (C - 1) % 3
        pltpu.make_async_copy(rows.at[bl2], acc.at[didx3.at[bl3]],
                              ssem).wait()
        plsc.subcore_barrier()
        pltpu.sync_copy(acc.at[pl.ds(s * rpt, rpt)],
                        out_hbm.at[c, pl.ds(s * rpt, rpt)])

    f = pl.kernel(
        body,
        out_type=jax.ShapeDtypeStruct((2, npad, d), jnp.float32),
        mesh=mesh,
        compiler_params=pltpu.CompilerParams(use_tc_tiling_on_sc=False),
        scratch_types=[
            pltpu.VMEM((3, ch), jnp.int32),
            pltpu.VMEM((3, ch), jnp.int32),
            pltpu.VMEM((2, ch, d), jnp.float32),
            pltpu.VMEM_SHARED((npad, d), jnp.float32),
            pltpu.SemaphoreType.DMA,
            pltpu.SemaphoreType.DMA,
            pltpu.SemaphoreType.DMA,
        ],
    )
    return f(u, src3, dst3)


def _mm_body(x_ref, w_ref, o_ref):
    o_ref[...] = jnp.dot(x_ref[...], w_ref[...],
                         preferred_element_type=jnp.float32)


def _tc_matmul(x, w):
    n, k = x.shape
    m = w.shape[1]
    return pl.pallas_call(
        _mm_body,
        grid=(n // _RB,),
        in_specs=[
            pl.BlockSpec((_RB, k), lambda i: (i, 0)),
            pl.BlockSpec((k, m), lambda i: (0, 0)),
        ],
        out_specs=pl.BlockSpec((_RB, m), lambda i: (i, 0)),
        out_shape=jax.ShapeDtypeStruct((n, m), jnp.float32),
    )(x, w)


def _scale_body(h_ref, a_ref, b_ref, o_ref):
    dinv = lax.rsqrt(1.0 + a_ref[:, 0:1] + b_ref[:, 0:1])
    o_ref[...] = h_ref[...] * dinv


def _tc_scale(h, d0, d1):
    n, m = h.shape
    return pl.pallas_call(
        _scale_body,
        grid=(n // _RB,),
        in_specs=[
            pl.BlockSpec((_RB, m), lambda i: (i, 0)),
            pl.BlockSpec((_RB, 16), lambda i: (i, 0)),
            pl.BlockSpec((_RB, 16), lambda i: (i, 0)),
        ],
        out_specs=pl.BlockSpec((_RB, m), lambda i: (i, 0)),
        out_shape=jax.ShapeDtypeStruct((n, m), jnp.float32),
    )(h, d0, d1)


def _l2_body(p0_ref, p1_ref, u1_ref, a_ref, b_ref, b1_ref, w2_ref, o_ref):
    dinv = lax.rsqrt(1.0 + a_ref[:, 0:1] + b_ref[:, 0:1])
    t = dinv * (p0_ref[...] + p1_ref[...] + u1_ref[...]) + b1_ref[...]
    t = jnp.maximum(t, 0.0)
    o_ref[...] = dinv * jnp.dot(t, w2_ref[...],
                                preferred_element_type=jnp.float32)


def _tc_layer2(p0, p1, u1, d0, d1, b1, w2):
    n, m = u1.shape
    ncls = w2.shape[1]
    return pl.pallas_call(
        _l2_body,
        grid=(n // _RB,),
        in_specs=[
            pl.BlockSpec((_RB, m), lambda i: (i, 0)),
            pl.BlockSpec((_RB, m), lambda i: (i, 0)),
            pl.BlockSpec((_RB, m), lambda i: (i, 0)),
            pl.BlockSpec((_RB, 16), lambda i: (i, 0)),
            pl.BlockSpec((_RB, 16), lambda i: (i, 0)),
            pl.BlockSpec((1, m), lambda i: (0, 0)),
            pl.BlockSpec((m, ncls), lambda i: (0, 0)),
        ],
        out_specs=pl.BlockSpec((_RB, ncls), lambda i: (i, 0)),
        out_shape=jax.ShapeDtypeStruct((n, ncls), jnp.float32),
    )(p0, p1, u1, d0, d1, b1, w2)


def _fin_body(q0_ref, q1_ref, u2_ref, a_ref, b_ref, b2_ref, o_ref):
    dinv = lax.rsqrt(1.0 + a_ref[:, 0:1] + b_ref[:, 0:1])
    z = dinv * (q0_ref[...] + q1_ref[...] + u2_ref[...]) + b2_ref[...]
    z = z - jnp.max(z, axis=1, keepdims=True)
    e = jnp.exp(z)
    o_ref[...] = e / jnp.sum(e, axis=1, keepdims=True)


def _tc_final(q0, q1, u2, d0, d1, b2):
    n, ncls = u2.shape
    return pl.pallas_call(
        _fin_body,
        grid=(n // _RB,),
        in_specs=[
            pl.BlockSpec((_RB, ncls), lambda i: (i, 0)),
            pl.BlockSpec((_RB, ncls), lambda i: (i, 0)),
            pl.BlockSpec((_RB, ncls), lambda i: (i, 0)),
            pl.BlockSpec((_RB, 16), lambda i: (i, 0)),
            pl.BlockSpec((_RB, 16), lambda i: (i, 0)),
            pl.BlockSpec((1, ncls), lambda i: (0, 0)),
        ],
        out_specs=pl.BlockSpec((_RB, ncls), lambda i: (i, 0)),
        out_shape=jax.ShapeDtypeStruct((n, ncls), jnp.float32),
    )(q0, q1, u2, d0, d1, b2)


def kernel(x, edge_index, W1, b1, W2, b2):
    n, _ = x.shape
    e = edge_index.shape[1]

    C = _cdiv(e, _NSLAB * _CH)
    epad = _NSLAB * C * _CH
    npad = (n // (16 * _CH) + 1) * (16 * _CH)

    pad = epad - e
    src = edge_index[0]
    dst = edge_index[1]
    src3 = jnp.concatenate(
        [src, jnp.zeros((pad,), jnp.int32)]).reshape(_NSLAB, C, _CH)
    dst3 = jnp.concatenate(
        [dst, jnp.full((pad,), n, jnp.int32)]).reshape(_NSLAB, C, _CH)

    deg = _sc_degree(dst3, npad)
    d0 = deg[0, :n]
    d1 = deg[1, :n]

    h1 = _tc_matmul(x, W1)
    u1 = _tc_scale(h1, d0, d1)

    p = _sc_agg(u1, src3, dst3, npad)
    u2 = _tc_layer2(p[0, :n], p[1, :n], u1, d0, d1,
                    b1.reshape(1, -1), W2)

    q = _sc_agg(u2, src3, dst3, npad)
    return _tc_final(q[0, :n], q[1, :n], u2, d0, d1, b2.reshape(1, -1))


# speedup vs baseline: 1.3143x; 1.0669x over previous
import functools

import jax
import jax.numpy as jnp
from jax import lax
from jax.experimental import pallas as pl
from jax.experimental.pallas import tpu as pltpu
from jax.experimental.pallas import tpu_sc as plsc

_CH = 128
_NSLAB = 32
_RB = 2000


def _cdiv(a, b):
    return (a + b - 1) // b


def _fill_const(ref, rows, d, val):
    vec = jnp.full((16,), val, jnp.float32)

    def row(i, carry):
        for jj in range(d // 16):
            ref[i, pl.ds(jj * 16, 16)] = vec
        return carry

    lax.fori_loop(0, rows, row, 0)


def _sc_degree(dst3, npad):
    nslab, C, ch = dst3.shape
    rpt = npad // 16
    mesh = plsc.VectorSubcoreMesh(core_axis_name="c", subcore_axis_name="s")

    def body(dst_hbm, out_hbm, didx, obuf, zbuf, acc):
        c = lax.axis_index("c")
        s = lax.axis_index("s")
        slab = c * 16 + s
        pltpu.sync_copy(dst_hbm.at[slab], didx)
        _fill_const(obuf, ch, 16, 1.0)
        _fill_const(zbuf, ch, 16, 0.0)
        for t in range(rpt // ch):
            pltpu.sync_copy(zbuf, acc.at[pl.ds(s * rpt + t * ch, ch)])
        plsc.subcore_barrier()

        def step(j, carry):
            pltpu.sync_copy(obuf, acc.at[didx.at[j]], add=True)
            return carry

        lax.fori_loop(0, C, step, 0)
        plsc.subcore_barrier()
        pltpu.sync_copy(acc.at[pl.ds(s * rpt, rpt)],
                        out_hbm.at[c, pl.ds(s * rpt, rpt)])

    f = pl.kernel(
        body,
        out_type=jax.ShapeDtypeStruct((2, npad, 16), jnp.float32),
        mesh=mesh,
        compiler_params=pltpu.CompilerParams(use_tc_tiling_on_sc=False),
        scratch_types=[
            pltpu.VMEM((C, ch), jnp.int32),
            pltpu.VMEM((ch, 16), jnp.float32),
            pltpu.VMEM((ch, 16), jnp.float32),
            pltpu.VMEM_SHARED((npad, 16), jnp.float32),
        ],
    )
    return f(dst3)


def _sc_agg(u, src2, dst2, npad, t0, t1):
    n, d = u.shape
    tch, ch = src2.shape
    assert 16 * (t0 + t1) == tch
    rpt = npad // 16
    mesh = plsc.VectorSubcoreMesh(core_axis_name="c", subcore_axis_name="s")

    def body(u_hbm, src_hbm, dst_hbm, out_hbm, sidx3, didx3, rows, acc,
             isem, gsem, ssem):
        c = lax.axis_index("c")
        s = lax.axis_index("s")
        C = jnp.where(c == 0, t0, t1)
        base = jnp.where(c == 0, s * t0, 16 * t0 + s * t1)
        _fill_const(rows.at[0], ch, d, 0.0)
        for t in range(rpt // ch):
            pltpu.sync_copy(rows.at[0], acc.at[pl.ds(s * rpt + t * ch, ch)])
        plsc.subcore_barrier()

        def load_idx(j, slot, sem):
            pltpu.async_copy(src_hbm.at[base + j], sidx3.at[slot], sem)
            pltpu.async_copy(dst_hbm.at[base + j], didx3.at[slot], sem)

        def wait_idx(j, slot, sem):
            pltpu.make_async_copy(src_hbm.at[base + j], sidx3.at[slot],
                                  sem).wait()
            pltpu.make_async_copy(dst_hbm.at[base + j], didx3.at[slot],
                                  sem).wait()

        load_idx(0, 0, isem)
        wait_idx(0, 0, isem)
        pltpu.async_copy(u_hbm.at[sidx3.at[0]], rows.at[0], gsem)
        load_idx(1, 1, isem)

        def step(j, carry):
            b2 = j % 2
            b3 = j % 3
            pltpu.make_async_copy(u_hbm.at[sidx3.at[b3]], rows.at[b2],
                                  gsem).wait()

            @pl.when(j > 0)
            def _():
                pltpu.make_async_copy(rows.at[1 - b2],
                                      acc.at[didx3.at[(j + 2) % 3]],
                                      ssem).wait()

            pltpu.async_copy(rows.at[b2], acc.at[didx3.at[b3]], ssem,
                             add=True)

            @pl.when(j + 1 < C)
            def _():
                wait_idx(j + 1, (j + 1) % 3, isem)
                pltpu.async_copy(u_hbm.at[sidx3.at[(j + 1) % 3]],
                                 rows.at[1 - b2], gsem)

            @pl.when(j + 2 < C)
            def _():
                load_idx(j + 2, (j + 2) % 3, isem)

            return carry

        lax.fori_loop(0, C, step, 0)
        bl2 = (C - 1) % 2
        bl3 = (C - 1) % 3
        pltpu.make_async_copy(rows.at[bl2], acc.at[didx3.at[bl3]],
                              ssem).wait()
        plsc.subcore_barrier()
        pltpu.sync_copy(acc.at[pl.ds(s * rpt, rpt)],
                        out_hbm.at[c, pl.ds(s * rpt, rpt)])

    f = pl.kernel(
        body,
        out_type=jax.ShapeDtypeStruct((2, npad, d), jnp.float32),
        mesh=mesh,
        compiler_params=pltpu.CompilerParams(use_tc_tiling_on_sc=False),
        scratch_types=[
            pltpu.VMEM((3, ch), jnp.int32),
            pltpu.VMEM((3, ch), jnp.int32),
            pltpu.VMEM((2, ch, d), jnp.float32),
            pltpu.VMEM_SHARED((npad, d), jnp.float32),
            pltpu.SemaphoreType.DMA,
            pltpu.SemaphoreType.DMA,
            pltpu.SemaphoreType.DMA,
        ],
    )
    return f(u, src2, dst2)


def _mm_body(x_ref, w_ref, o_ref):
    o_ref[...] = jnp.dot(x_ref[...], w_ref[...],
                         preferred_element_type=jnp.float32)


def _tc_matmul(x, w):
    n, k = x.shape
    m = w.shape[1]
    return pl.pallas_call(
        _mm_body,
        grid=(n // _RB,),
        in_specs=[
            pl.BlockSpec((_RB, k), lambda i: (i, 0)),
            pl.BlockSpec((k, m), lambda i: (0, 0)),
        ],
        out_specs=pl.BlockSpec((_RB, m), lambda i: (i, 0)),
        out_shape=jax.ShapeDtypeStruct((n, m), jnp.float32),
    )(x, w)


def _scale_body(h_ref, a_ref, b_ref, o_ref):
    dinv = lax.rsqrt(1.0 + a_ref[:, 0:1] + b_ref[:, 0:1])
    o_ref[...] = h_ref[...] * dinv


def _tc_scale(h, d0, d1):
    n, m = h.shape
    return pl.pallas_call(
        _scale_body,
        grid=(n // _RB,),
        in_specs=[
            pl.BlockSpec((_RB, m), lambda i: (i, 0)),
            pl.BlockSpec((_RB, 16), lambda i: (i, 0)),
            pl.BlockSpec((_RB, 16), lambda i: (i, 0)),
        ],
        out_specs=pl.BlockSpec((_RB, m), lambda i: (i, 0)),
        out_shape=jax.ShapeDtypeStruct((n, m), jnp.float32),
    )(h, d0, d1)


def _l2_body(p0_ref, p1_ref, u1_ref, a_ref, b_ref, b1_ref, w2_ref, o_ref):
    dinv = lax.rsqrt(1.0 + a_ref[:, 0:1] + b_ref[:, 0:1])
    t = dinv * (p0_ref[...] + p1_ref[...] + u1_ref[...]) + b1_ref[...]
    t = jnp.maximum(t, 0.0)
    o_ref[...] = dinv * jnp.dot(t, w2_ref[...],
                                preferred_element_type=jnp.float32)


def _tc_layer2(p0, p1, u1, d0, d1, b1, w2):
    n, m = u1.shape
    ncls = w2.shape[1]
    return pl.pallas_call(
        _l2_body,
        grid=(n // _RB,),
        in_specs=[
            pl.BlockSpec((_RB, m), lambda i: (i, 0)),
            pl.BlockSpec((_RB, m), lambda i: (i, 0)),
            pl.BlockSpec((_RB, m), lambda i: (i, 0)),
            pl.BlockSpec((_RB, 16), lambda i: (i, 0)),
            pl.BlockSpec((_RB, 16), lambda i: (i, 0)),
            pl.BlockSpec((1, m), lambda i: (0, 0)),
            pl.BlockSpec((m, ncls), lambda i: (0, 0)),
        ],
        out_specs=pl.BlockSpec((_RB, ncls), lambda i: (i, 0)),
        out_shape=jax.ShapeDtypeStruct((n, ncls), jnp.float32),
    )(p0, p1, u1, d0, d1, b1, w2)


def _fin_body(q0_ref, q1_ref, u2_ref, a_ref, b_ref, b2_ref, o_ref):
    dinv = lax.rsqrt(1.0 + a_ref[:, 0:1] + b_ref[:, 0:1])
    z = dinv * (q0_ref[...] + q1_ref[...] + u2_ref[...]) + b2_ref[...]
    z = z - jnp.max(z, axis=1, keepdims=True)
    e = jnp.exp(z)
    o_ref[...] = e / jnp.sum(e, axis=1, keepdims=True)


def _tc_final(q0, q1, u2, d0, d1, b2):
    n, ncls = u2.shape
    return pl.pallas_call(
        _fin_body,
        grid=(n // _RB,),
        in_specs=[
            pl.BlockSpec((_RB, ncls), lambda i: (i, 0)),
            pl.BlockSpec((_RB, ncls), lambda i: (i, 0)),
            pl.BlockSpec((_RB, ncls), lambda i: (i, 0)),
            pl.BlockSpec((_RB, 16), lambda i: (i, 0)),
            pl.BlockSpec((_RB, 16), lambda i: (i, 0)),
            pl.BlockSpec((1, ncls), lambda i: (0, 0)),
        ],
        out_specs=pl.BlockSpec((_RB, ncls), lambda i: (i, 0)),
        out_shape=jax.ShapeDtypeStruct((n, ncls), jnp.float32),
    )(q0, q1, u2, d0, d1, b2)


def kernel(x, edge_index, W1, b1, W2, b2):
    n, _ = x.shape
    e = edge_index.shape[1]

    C = _cdiv(e, _NSLAB * _CH)
    epad = _NSLAB * C * _CH
    npad = (n // (16 * _CH) + 1) * (16 * _CH)

    pad = epad - e
    src = edge_index[0]
    dst = edge_index[1]
    src2 = jnp.concatenate(
        [src, jnp.zeros((pad,), jnp.int32)]).reshape(_NSLAB * C, _CH)
    dst2 = jnp.concatenate(
        [dst, jnp.full((pad,), n, jnp.int32)]).reshape(_NSLAB * C, _CH)

    tot = _NSLAB * C // 16
    t0 = (tot * 110 + 79) // 158
    t1 = tot - t0

    deg = _sc_degree(dst2.reshape(_NSLAB, C, _CH), npad)
    d0 = deg[0, :n]
    d1 = deg[1, :n]

    h1 = _tc_matmul(x, W1)
    u1 = _tc_scale(h1, d0, d1)

    p = _sc_agg(u1, src2, dst2, npad, t0, t1)
    u2 = _tc_layer2(p[0, :n], p[1, :n], u1, d0, d1,
                    b1.reshape(1, -1), W2)

    q = _sc_agg(u2, src2, dst2, npad, t0, t1)
    return _tc_final(q[0, :n], q[1, :n], u2, d0, d1, b2.reshape(1, -1))


# speedup vs baseline: 1.6305x; 1.2406x over previous
import functools

import jax
import jax.numpy as jnp
from jax import lax
from jax.experimental import pallas as pl
from jax.experimental.pallas import tpu as pltpu
from jax.experimental.pallas import tpu_sc as plsc

_CH = 128
_NSLAB = 32
_RB = 2000


def _cdiv(a, b):
    return (a + b - 1) // b


def _fill_const(ref, rows, d, val):
    vec = jnp.full((16,), val, jnp.float32)

    def row(i, carry):
        for jj in range(d // 16):
            ref[i, pl.ds(jj * 16, 16)] = vec
        return carry

    lax.fori_loop(0, rows, row, 0)


def _sc_degree(dst3, npad):
    nslab, C, ch = dst3.shape
    rpt = npad // 16
    mesh = plsc.VectorSubcoreMesh(core_axis_name="c", subcore_axis_name="s")

    def body(dst_hbm, out_hbm, didx, obuf, zbuf, acc):
        c = lax.axis_index("c")
        s = lax.axis_index("s")
        slab = c * 16 + s
        pltpu.sync_copy(dst_hbm.at[slab], didx)
        _fill_const(obuf, ch, 16, 1.0)
        _fill_const(zbuf, ch, 16, 0.0)
        for t in range(rpt // ch):
            pltpu.sync_copy(zbuf, acc.at[pl.ds(s * rpt + t * ch, ch)])
        plsc.subcore_barrier()

        def step(j, carry):
            pltpu.sync_copy(obuf, acc.at[didx.at[j]], add=True)
            return carry

        lax.fori_loop(0, C, step, 0)
        plsc.subcore_barrier()
        pltpu.sync_copy(acc.at[pl.ds(s * rpt, rpt)],
                        out_hbm.at[c, pl.ds(s * rpt, rpt)])

    f = pl.kernel(
        body,
        out_type=jax.ShapeDtypeStruct((2, npad, 16), jnp.float32),
        mesh=mesh,
        compiler_params=pltpu.CompilerParams(use_tc_tiling_on_sc=False),
        scratch_types=[
            pltpu.VMEM((C, ch), jnp.int32),
            pltpu.VMEM((ch, 16), jnp.float32),
            pltpu.VMEM((ch, 16), jnp.float32),
            pltpu.VMEM_SHARED((npad, 16), jnp.float32),
        ],
    )
    return f(dst3)


def _sc_agg128(uh, src2, dst2, npad):
    _, n, d = uh.shape
    tch, ch = src2.shape
    cpt = tch // 16
    rpt = npad // 16
    nrt = n // 16
    t_half = tch // 32
    mesh = plsc.VectorSubcoreMesh(core_axis_name="c", subcore_axis_name="s")

    def body(u_hbm, src_hbm, dst_hbm, out_hbm, sidx3, didx3, rows, ushr,
             acc, isem, gsem, ssem):
        c = lax.axis_index("c")
        s = lax.axis_index("s")
        base = (c * 16 + s) * t_half
        C = t_half
        _fill_const(rows.at[0], ch, d, 0.0)

        def load_idx(j, slot, sem):
            pltpu.async_copy(src_hbm.at[base + j], sidx3.at[slot], sem)
            pltpu.async_copy(dst_hbm.at[base + j], didx3.at[slot], sem)

        def wait_idx(j, slot, sem):
            pltpu.make_async_copy(src_hbm.at[base + j], sidx3.at[slot],
                                  sem).wait()
            pltpu.make_async_copy(dst_hbm.at[base + j], didx3.at[slot],
                                  sem).wait()

        for h in range(2):
            pltpu.sync_copy(u_hbm.at[h, pl.ds(s * nrt, nrt)],
                            ushr.at[pl.ds(s * nrt, nrt)])
            for t in range(rpt // ch):
                pltpu.sync_copy(rows.at[0],
                                acc.at[pl.ds(s * rpt + t * ch, ch)])
            plsc.subcore_barrier()

            load_idx(0, 0, isem)
            wait_idx(0, 0, isem)
            pltpu.async_copy(ushr.at[sidx3.at[0]], rows.at[0], gsem)
            load_idx(1, 1, isem)

            def step(j, carry):
                b2 = j % 2
                b3 = j % 3
                pltpu.make_async_copy(ushr.at[sidx3.at[b3]], rows.at[b2],
                                      gsem).wait()

                @pl.when(j > 0)
                def _():
                    pltpu.make_async_copy(rows.at[1 - b2],
                                          acc.at[didx3.at[(j + 2) % 3]],
                                          ssem).wait()

                pltpu.async_copy(rows.at[b2], acc.at[didx3.at[b3]], ssem,
                                 add=True)

                @pl.when(j + 1 < C)
                def _():
                    wait_idx(j + 1, (j + 1) % 3, isem)
                    pltpu.async_copy(ushr.at[sidx3.at[(j + 1) % 3]],
                                     rows.at[1 - b2], gsem)

                @pl.when(j + 2 < C)
                def _():
                    load_idx(j + 2, (j + 2) % 3, isem)

                return carry

            lax.fori_loop(0, C, step, 0)
            bl2 = (C - 1) % 2
            bl3 = (C - 1) % 3
            pltpu.make_async_copy(rows.at[bl2], acc.at[didx3.at[bl3]],
                                  ssem).wait()
            plsc.subcore_barrier()
            pltpu.sync_copy(acc.at[pl.ds(s * rpt, rpt)],
                            out_hbm.at[c, h, pl.ds(s * rpt, rpt)])
            _fill_const(rows.at[0], ch, d, 0.0)

    f = pl.kernel(
        body,
        out_type=jax.ShapeDtypeStruct((2, 2, npad, d), jnp.float32),
        mesh=mesh,
        compiler_params=pltpu.CompilerParams(use_tc_tiling_on_sc=False),
        scratch_types=[
            pltpu.VMEM((3, ch), jnp.int32),
            pltpu.VMEM((3, ch), jnp.int32),
            pltpu.VMEM((2, ch, d), jnp.float32),
            pltpu.VMEM_SHARED((n, d), jnp.float32),
            pltpu.VMEM_SHARED((npad, d), jnp.float32),
            pltpu.SemaphoreType.DMA,
            pltpu.SemaphoreType.DMA,
            pltpu.SemaphoreType.DMA,
        ],
    )
    return f(uh, src2, dst2)


def _sc_agg(u, src2, dst2, npad, t0, t1):
    n, d = u.shape
    tch, ch = src2.shape
    assert 16 * (t0 + t1) == tch
    rpt = npad // 16
    mesh = plsc.VectorSubcoreMesh(core_axis_name="c", subcore_axis_name="s")

    def body(u_hbm, src_hbm, dst_hbm, out_hbm, sidx3, didx3, rows, acc,
             isem, gsem, ssem):
        c = lax.axis_index("c")
        s = lax.axis_index("s")
        C = jnp.where(c == 0, t0, t1)
        base = jnp.where(c == 0, s * t0, 16 * t0 + s * t1)
        _fill_const(rows.at[0], ch, d, 0.0)
        for t in range(rpt // ch):
            pltpu.sync_copy(rows.at[0], acc.at[pl.ds(s * rpt + t * ch, ch)])
        plsc.subcore_barrier()

        def load_idx(j, slot, sem):
            pltpu.async_copy(src_hbm.at[base + j], sidx3.at[slot], sem)
            pltpu.async_copy(dst_hbm.at[base + j], didx3.at[slot], sem)

        def wait_idx(j, slot, sem):
            pltpu.make_async_copy(src_hbm.at[base + j], sidx3.at[slot],
                                  sem).wait()
            pltpu.make_async_copy(dst_hbm.at[base + j], didx3.at[slot],
                                  sem).wait()

        load_idx(0, 0, isem)
        wait_idx(0, 0, isem)
        pltpu.async_copy(u_hbm.at[sidx3.at[0]], rows.at[0], gsem)
        load_idx(1, 1, isem)

        def step(j, carry):
            b2 = j % 2
            b3 = j % 3
            pltpu.make_async_copy(u_hbm.at[sidx3.at[b3]], rows.at[b2],
                                  gsem).wait()

            @pl.when(j > 0)
            def _():
                pltpu.make_async_copy(rows.at[1 - b2],
                                      acc.at[didx3.at[(j + 2) % 3]],
                                      ssem).wait()

            pltpu.async_copy(rows.at[b2], acc.at[didx3.at[b3]], ssem,
                             add=True)

            @pl.when(j + 1 < C)
            def _():
                wait_idx(j + 1, (j + 1) % 3, isem)
                pltpu.async_copy(u_hbm.at[sidx3.at[(j + 1) % 3]],
                                 rows.at[1 - b2], gsem)

            @pl.when(j + 2 < C)
            def _():
                load_idx(j + 2, (j + 2) % 3, isem)

            return carry

        lax.fori_loop(0, C, step, 0)
        bl2 = (C - 1) % 2
        bl3 = (C - 1) % 3
        pltpu.make_async_copy(rows.at[bl2], acc.at[didx3.at[bl3]],
                              ssem).wait()
        plsc.subcore_barrier()
        pltpu.sync_copy(acc.at[pl.ds(s * rpt, rpt)],
                        out_hbm.at[c, pl.ds(s * rpt, rpt)])

    f = pl.kernel(
        body,
        out_type=jax.ShapeDtypeStruct((2, npad, d), jnp.float32),
        mesh=mesh,
        compiler_params=pltpu.CompilerParams(use_tc_tiling_on_sc=False),
        scratch_types=[
            pltpu.VMEM((3, ch), jnp.int32),
            pltpu.VMEM((3, ch), jnp.int32),
            pltpu.VMEM((2, ch, d), jnp.float32),
            pltpu.VMEM_SHARED((npad, d), jnp.float32),
            pltpu.SemaphoreType.DMA,
            pltpu.SemaphoreType.DMA,
            pltpu.SemaphoreType.DMA,
        ],
    )
    return f(u, src2, dst2)


def _mm_body(x_ref, w_ref, o_ref):
    o_ref[...] = jnp.dot(x_ref[...], w_ref[...],
                         preferred_element_type=jnp.float32)


def _tc_matmul(x, w):
    n, k = x.shape
    m = w.shape[1]
    return pl.pallas_call(
        _mm_body,
        grid=(n // _RB,),
        in_specs=[
            pl.BlockSpec((_RB, k), lambda i: (i, 0)),
            pl.BlockSpec((k, m), lambda i: (0, 0)),
        ],
        out_specs=pl.BlockSpec((_RB, m), lambda i: (i, 0)),
        out_shape=jax.ShapeDtypeStruct((n, m), jnp.float32),
    )(x, w)


def _scale_body(h_ref, a_ref, b_ref, o_ref):
    hw = h_ref.shape[1] // 2
    dinv = lax.rsqrt(1.0 + a_ref[:, 0:1] + b_ref[:, 0:1])
    u = h_ref[...] * dinv
    o_ref[0] = u[:, :hw]
    o_ref[1] = u[:, hw:]


def _tc_scale(h, d0, d1):
    n, m = h.shape
    hw = m // 2
    return pl.pallas_call(
        _scale_body,
        grid=(n // _RB,),
        in_specs=[
            pl.BlockSpec((_RB, m), lambda i: (i, 0)),
            pl.BlockSpec((_RB, 16), lambda i: (i, 0)),
            pl.BlockSpec((_RB, 16), lambda i: (i, 0)),
        ],
        out_specs=pl.BlockSpec((2, _RB, hw), lambda i: (0, i, 0)),
        out_shape=jax.ShapeDtypeStruct((2, n, hw), jnp.float32),
    )(h, d0, d1)


def _l2_body(pa0_ref, pb0_ref, pa1_ref, pb1_ref, ua_ref, ub_ref,
             a_ref, b_ref, b1a_ref, b1b_ref, w2a_ref, w2b_ref, o_ref):
    dinv = lax.rsqrt(1.0 + a_ref[:, 0:1] + b_ref[:, 0:1])
    t0 = dinv * (pa0_ref[0, 0] + pb0_ref[0, 0] + ua_ref[0]) + b1a_ref[0]
    t1 = dinv * (pa1_ref[0, 0] + pb1_ref[0, 0] + ub_ref[0]) + b1b_ref[0]
    t0 = jnp.maximum(t0, 0.0)
    t1 = jnp.maximum(t1, 0.0)
    o_ref[...] = dinv * (
        jnp.dot(t0, w2a_ref[...], preferred_element_type=jnp.float32)
        + jnp.dot(t1, w2b_ref[...], preferred_element_type=jnp.float32))


def _tc_layer2(p4, uh, d0, d1, b1, w2):
    _, n, hw = uh.shape
    ncls = w2.shape[1]
    psp = lambda cc, hh: pl.BlockSpec((1, 1, _RB, hw),
                                      lambda i: (cc, hh, i, 0))
    return pl.pallas_call(
        _l2_body,
        grid=(n // _RB,),
        in_specs=[
            psp(0, 0), psp(1, 0), psp(0, 1), psp(1, 1),
            pl.BlockSpec((1, _RB, hw), lambda i: (0, i, 0)),
            pl.BlockSpec((1, _RB, hw), lambda i: (1, i, 0)),
            pl.BlockSpec((_RB, 16), lambda i: (i, 0)),
            pl.BlockSpec((_RB, 16), lambda i: (i, 0)),
            pl.BlockSpec((1, 1, hw), lambda i: (0, 0, 0)),
            pl.BlockSpec((1, 1, hw), lambda i: (1, 0, 0)),
            pl.BlockSpec((hw, ncls), lambda i: (0, 0)),
            pl.BlockSpec((hw, ncls), lambda i: (1, 0)),
        ],
        out_specs=pl.BlockSpec((_RB, ncls), lambda i: (i, 0)),
        out_shape=jax.ShapeDtypeStruct((n, ncls), jnp.float32),
    )(p4, p4, p4, p4, uh, uh, d0, d1, b1, b1, w2, w2)


def _fin_body(q0_ref, q1_ref, u2_ref, a_ref, b_ref, b2_ref, o_ref):
    dinv = lax.rsqrt(1.0 + a_ref[:, 0:1] + b_ref[:, 0:1])
    z = dinv * (q0_ref[...] + q1_ref[...] + u2_ref[...]) + b2_ref[...]
    z = z - jnp.max(z, axis=1, keepdims=True)
    e = jnp.exp(z)
    o_ref[...] = e / jnp.sum(e, axis=1, keepdims=True)


def _tc_final(q0, q1, u2, d0, d1, b2):
    n, ncls = u2.shape
    return pl.pallas_call(
        _fin_body,
        grid=(n // _RB,),
        in_specs=[
            pl.BlockSpec((_RB, ncls), lambda i: (i, 0)),
            pl.BlockSpec((_RB, ncls), lambda i: (i, 0)),
            pl.BlockSpec((_RB, ncls), lambda i: (i, 0)),
            pl.BlockSpec((_RB, 16), lambda i: (i, 0)),
            pl.BlockSpec((_RB, 16), lambda i: (i, 0)),
            pl.BlockSpec((1, ncls), lambda i: (0, 0)),
        ],
        out_specs=pl.BlockSpec((_RB, ncls), lambda i: (i, 0)),
        out_shape=jax.ShapeDtypeStruct((n, ncls), jnp.float32),
    )(q0, q1, u2, d0, d1, b2)


def kernel(x, edge_index, W1, b1, W2, b2):
    n, _ = x.shape
    e = edge_index.shape[1]

    C = _cdiv(e, _NSLAB * _CH)
    epad = _NSLAB * C * _CH
    npad = (n // (16 * _CH) + 1) * (16 * _CH)

    pad = epad - e
    src = edge_index[0]
    dst = edge_index[1]
    src2 = jnp.concatenate(
        [src, jnp.zeros((pad,), jnp.int32)]).reshape(_NSLAB * C, _CH)
    dst2 = jnp.concatenate(
        [dst, jnp.full((pad,), n, jnp.int32)]).reshape(_NSLAB * C, _CH)

    tot = _NSLAB * C // 16
    t0 = (tot * 110 + 79) // 158
    t1 = tot - t0

    deg = _sc_degree(dst2.reshape(_NSLAB, C, _CH), npad)
    d0 = deg[0, :n]
    d1 = deg[1, :n]

    h1 = _tc_matmul(x, W1)
    u1h = _tc_scale(h1, d0, d1)

    p4 = _sc_agg128(u1h, src2, dst2, npad)
    u2 = _tc_layer2(p4, u1h, d0, d1, b1.reshape(2, 1, -1), W2)

    q = _sc_agg(u2, src2, dst2, npad, t0, t1)
    return _tc_final(q[0, :n], q[1, :n], u2, d0, d1, b2.reshape(1, -1))


# speedup vs baseline: 1.9211x; 1.1782x over previous
import functools

import jax
import jax.numpy as jnp
from jax import lax
from jax.experimental import pallas as pl
from jax.experimental.pallas import tpu as pltpu
from jax.experimental.pallas import tpu_sc as plsc

_CH = 128
_NSLAB = 32
_RB = 2000


def _cdiv(a, b):
    return (a + b - 1) // b


def _fill_const(ref, rows, d, val):
    vec = jnp.full((16,), val, jnp.float32)

    def row(i, carry):
        for jj in range(d // 16):
            ref[i, pl.ds(jj * 16, 16)] = vec
        return carry

    lax.fori_loop(0, rows, row, 0)


def _sc_degree(dst3, npad):
    nslab, C, ch = dst3.shape
    rpt = npad // 16
    mesh = plsc.VectorSubcoreMesh(core_axis_name="c", subcore_axis_name="s")

    def body(dst_hbm, out_hbm, didx, obuf, zbuf, acc):
        c = lax.axis_index("c")
        s = lax.axis_index("s")
        slab = c * 16 + s
        pltpu.sync_copy(dst_hbm.at[slab], didx)
        _fill_const(obuf, ch, 16, 1.0)
        _fill_const(zbuf, ch, 16, 0.0)
        for t in range(rpt // ch):
            pltpu.sync_copy(zbuf, acc.at[pl.ds(s * rpt + t * ch, ch)])
        plsc.subcore_barrier()

        def step(j, carry):
            pltpu.sync_copy(obuf, acc.at[didx.at[j]], add=True)
            return carry

        lax.fori_loop(0, C, step, 0)
        plsc.subcore_barrier()
        pltpu.sync_copy(acc.at[pl.ds(s * rpt, rpt)],
                        out_hbm.at[c, pl.ds(s * rpt, rpt)])

    f = pl.kernel(
        body,
        out_type=jax.ShapeDtypeStruct((2, npad, 16), jnp.float32),
        mesh=mesh,
        compiler_params=pltpu.CompilerParams(use_tc_tiling_on_sc=False),
        scratch_types=[
            pltpu.VMEM((C, ch), jnp.int32),
            pltpu.VMEM((ch, 16), jnp.float32),
            pltpu.VMEM((ch, 16), jnp.float32),
            pltpu.VMEM_SHARED((npad, 16), jnp.float32),
        ],
    )
    return f(dst3)


def _sc_agg128(uh, src2, dst2, npad):
    _, n, d = uh.shape
    tch, ch = src2.shape
    cpt = tch // 16
    rpt = npad // 16
    nrt = n // 16
    t_half = tch // 32
    mesh = plsc.VectorSubcoreMesh(core_axis_name="c", subcore_axis_name="s")

    def body(u_hbm, src_hbm, dst_hbm, out_hbm, sidx3, didx3, rows, ushr,
             acc, isem, gsem, ssem):
        c = lax.axis_index("c")
        s = lax.axis_index("s")
        base = (c * 16 + s) * t_half
        C = t_half
        _fill_const(rows.at[0], ch, d, 0.0)

        def load_idx(j, slot, sem):
            pltpu.async_copy(src_hbm.at[base + j], sidx3.at[slot], sem)
            pltpu.async_copy(dst_hbm.at[base + j], didx3.at[slot], sem)

        def wait_idx(j, slot, sem):
            pltpu.make_async_copy(src_hbm.at[base + j], sidx3.at[slot],
                                  sem).wait()
            pltpu.make_async_copy(dst_hbm.at[base + j], didx3.at[slot],
                                  sem).wait()

        for h in range(2):
            pltpu.sync_copy(u_hbm.at[h, pl.ds(s * nrt, nrt)],
                            ushr.at[pl.ds(s * nrt, nrt)])
            for t in range(rpt // ch):
                pltpu.sync_copy(rows.at[0],
                                acc.at[pl.ds(s * rpt + t * ch, ch)])
            plsc.subcore_barrier()

            load_idx(0, 0, isem)
            wait_idx(0, 0, isem)
            pltpu.async_copy(ushr.at[sidx3.at[0]], rows.at[0], gsem)
            load_idx(1, 1, isem)

            def step(j, carry):
                b2 = j % 2
                b3 = j % 3
                pltpu.make_async_copy(ushr.at[sidx3.at[b3]], rows.at[b2],
                                      gsem).wait()

                @pl.when(j > 0)
                def _():
                    pltpu.make_async_copy(rows.at[1 - b2],
                                          acc.at[didx3.at[(j + 2) % 3]],
                                          ssem).wait()

                pltpu.async_copy(rows.at[b2], acc.at[didx3.at[b3]], ssem,
                                 add=True)

                @pl.when(j + 1 < C)
                def _():
                    wait_idx(j + 1, (j + 1) % 3, isem)
                    pltpu.async_copy(ushr.at[sidx3.at[(j + 1) % 3]],
                                     rows.at[1 - b2], gsem)

                @pl.when(j + 2 < C)
                def _():
                    load_idx(j + 2, (j + 2) % 3, isem)

                return carry

            lax.fori_loop(0, C, step, 0)
            bl2 = (C - 1) % 2
            bl3 = (C - 1) % 3
            pltpu.make_async_copy(rows.at[bl2], acc.at[didx3.at[bl3]],
                                  ssem).wait()
            plsc.subcore_barrier()
            pltpu.sync_copy(acc.at[pl.ds(s * rpt, rpt)],
                            out_hbm.at[c, h, pl.ds(s * rpt, rpt)])
            _fill_const(rows.at[0], ch, d, 0.0)

    f = pl.kernel(
        body,
        out_type=jax.ShapeDtypeStruct((2, 2, npad, d), jnp.float32),
        mesh=mesh,
        compiler_params=pltpu.CompilerParams(use_tc_tiling_on_sc=False),
        scratch_types=[
            pltpu.VMEM((3, ch), jnp.int32),
            pltpu.VMEM((3, ch), jnp.int32),
            pltpu.VMEM((2, ch, d), jnp.float32),
            pltpu.VMEM_SHARED((n, d), jnp.float32),
            pltpu.VMEM_SHARED((npad, d), jnp.float32),
            pltpu.SemaphoreType.DMA,
            pltpu.SemaphoreType.DMA,
            pltpu.SemaphoreType.DMA,
        ],
    )
    return f(uh, src2, dst2)


def _sc_agg_spmem(u, src2, dst2, npad):
    n, d = u.shape
    tch, ch = src2.shape
    rpt = npad // 16
    nrt = n // 16
    t_half = tch // 32
    mesh = plsc.VectorSubcoreMesh(core_axis_name="c", subcore_axis_name="s")

    def body(u_hbm, src_hbm, dst_hbm, out_hbm, sidx3, didx3, rows, ushr,
             acc, isem, gsem, ssem):
        c = lax.axis_index("c")
        s = lax.axis_index("s")
        base = (c * 16 + s) * t_half
        C = t_half
        pltpu.sync_copy(u_hbm.at[pl.ds(s * nrt, nrt)],
                        ushr.at[pl.ds(s * nrt, nrt)])
        _fill_const(rows.at[0], ch, d, 0.0)
        for t in range(rpt // ch):
            pltpu.sync_copy(rows.at[0], acc.at[pl.ds(s * rpt + t * ch, ch)])
        plsc.subcore_barrier()

        def load_idx(j, slot, sem):
            pltpu.async_copy(src_hbm.at[base + j], sidx3.at[slot], sem)
            pltpu.async_copy(dst_hbm.at[base + j], didx3.at[slot], sem)

        def wait_idx(j, slot, sem):
            pltpu.make_async_copy(src_hbm.at[base + j], sidx3.at[slot],
                                  sem).wait()
            pltpu.make_async_copy(dst_hbm.at[base + j], didx3.at[slot],
                                  sem).wait()

        load_idx(0, 0, isem)
        wait_idx(0, 0, isem)
        pltpu.async_copy(ushr.at[sidx3.at[0]], rows.at[0], gsem)
        load_idx(1, 1, isem)

        def step(j, carry):
            b2 = j % 2
            b3 = j % 3
            pltpu.make_async_copy(ushr.at[sidx3.at[b3]], rows.at[b2],
                                  gsem).wait()

            @pl.when(j > 0)
            def _():
                pltpu.make_async_copy(rows.at[1 - b2],
                                      acc.at[didx3.at[(j + 2) % 3]],
                                      ssem).wait()

            pltpu.async_copy(rows.at[b2], acc.at[didx3.at[b3]], ssem,
                             add=True)

            @pl.when(j + 1 < C)
            def _():
                wait_idx(j + 1, (j + 1) % 3, isem)
                pltpu.async_copy(ushr.at[sidx3.at[(j + 1) % 3]],
                                 rows.at[1 - b2], gsem)

            @pl.when(j + 2 < C)
            def _():
                load_idx(j + 2, (j + 2) % 3, isem)

            return carry

        lax.fori_loop(0, C, step, 0)
        bl2 = (C - 1) % 2
        bl3 = (C - 1) % 3
        pltpu.make_async_copy(rows.at[bl2], acc.at[didx3.at[bl3]],
                              ssem).wait()
        plsc.subcore_barrier()
        pltpu.sync_copy(acc.at[pl.ds(s * rpt, rpt)],
                        out_hbm.at[c, pl.ds(s * rpt, rpt)])

    f = pl.kernel(
        body,
        out_type=jax.ShapeDtypeStruct((2, npad, d), jnp.float32),
        mesh=mesh,
        compiler_params=pltpu.CompilerParams(use_tc_tiling_on_sc=False),
        scratch_types=[
            pltpu.VMEM((3, ch), jnp.int32),
            pltpu.VMEM((3, ch), jnp.int32),
            pltpu.VMEM((2, ch, d), jnp.float32),
            pltpu.VMEM_SHARED((n, d), jnp.float32),
            pltpu.VMEM_SHARED((npad, d), jnp.float32),
            pltpu.SemaphoreType.DMA,
            pltpu.SemaphoreType.DMA,
            pltpu.SemaphoreType.DMA,
        ],
    )
    return f(u, src2, dst2)


def _sc_agg(u, src2, dst2, npad, t0, t1):
    n, d = u.shape
    tch, ch = src2.shape
    assert 16 * (t0 + t1) == tch
    rpt = npad // 16
    mesh = plsc.VectorSubcoreMesh(core_axis_name="c", subcore_axis_name="s")

    def body(u_hbm, src_hbm, dst_hbm, out_hbm, sidx3, didx3, rows, acc,
             isem, gsem, ssem):
        c = lax.axis_index("c")
        s = lax.axis_index("s")
        C = jnp.where(c == 0, t0, t1)
        base = jnp.where(c == 0, s * t0, 16 * t0 + s * t1)
        _fill_const(rows.at[0], ch, d, 0.0)
        for t in range(rpt // ch):
            pltpu.sync_copy(rows.at[0], acc.at[pl.ds(s * rpt + t * ch, ch)])
        plsc.subcore_barrier()

        def load_idx(j, slot, sem):
            pltpu.async_copy(src_hbm.at[base + j], sidx3.at[slot], sem)
            pltpu.async_copy(dst_hbm.at[base + j], didx3.at[slot], sem)

        def wait_idx(j, slot, sem):
            pltpu.make_async_copy(src_hbm.at[base + j], sidx3.at[slot],
                                  sem).wait()
            pltpu.make_async_copy(dst_hbm.at[base + j], didx3.at[slot],
                                  sem).wait()

        load_idx(0, 0, isem)
        wait_idx(0, 0, isem)
        pltpu.async_copy(u_hbm.at[sidx3.at[0]], rows.at[0], gsem)
        load_idx(1, 1, isem)

        def step(j, carry):
            b2 = j % 2
            b3 = j % 3
            pltpu.make_async_copy(u_hbm.at[sidx3.at[b3]], rows.at[b2],
                                  gsem).wait()

            @pl.when(j > 0)
            def _():
                pltpu.make_async_copy(rows.at[1 - b2],
                                      acc.at[didx3.at[(j + 2) % 3]],
                                      ssem).wait()

            pltpu.async_copy(rows.at[b2], acc.at[didx3.at[b3]], ssem,
                             add=True)

            @pl.when(j + 1 < C)
            def _():
                wait_idx(j + 1, (j + 1) % 3, isem)
                pltpu.async_copy(u_hbm.at[sidx3.at[(j + 1) % 3]],
                                 rows.at[1 - b2], gsem)

            @pl.when(j + 2 < C)
            def _():
                load_idx(j + 2, (j + 2) % 3, isem)

            return carry

        lax.fori_loop(0, C, step, 0)
        bl2 = (C - 1) % 2
        bl3 = (C - 1) % 3
        pltpu.make_async_copy(rows.at[bl2], acc.at[didx3.at[bl3]],
                              ssem).wait()
        plsc.subcore_barrier()
        pltpu.sync_copy(acc.at[pl.ds(s * rpt, rpt)],
                        out_hbm.at[c, pl.ds(s * rpt, rpt)])

    f = pl.kernel(
        body,
        out_type=jax.ShapeDtypeStruct((2, npad, d), jnp.float32),
        mesh=mesh,
        compiler_params=pltpu.CompilerParams(use_tc_tiling_on_sc=False),
        scratch_types=[
            pltpu.VMEM((3, ch), jnp.int32),
            pltpu.VMEM((3, ch), jnp.int32),
            pltpu.VMEM((2, ch, d), jnp.float32),
            pltpu.VMEM_SHARED((npad, d), jnp.float32),
            pltpu.SemaphoreType.DMA,
            pltpu.SemaphoreType.DMA,
            pltpu.SemaphoreType.DMA,
        ],
    )
    return f(u, src2, dst2)


def _mm_body(x_ref, w_ref, o_ref):
    o_ref[...] = jnp.dot(x_ref[...], w_ref[...],
                         preferred_element_type=jnp.float32)


def _tc_matmul(x, w):
    n, k = x.shape
    m = w.shape[1]
    return pl.pallas_call(
        _mm_body,
        grid=(n // _RB,),
        in_specs=[
            pl.BlockSpec((_RB, k), lambda i: (i, 0)),
            pl.BlockSpec((k, m), lambda i: (0, 0)),
        ],
        out_specs=pl.BlockSpec((_RB, m), lambda i: (i, 0)),
        out_shape=jax.ShapeDtypeStruct((n, m), jnp.float32),
    )(x, w)


def _scale_body(h_ref, a_ref, b_ref, o_ref):
    hw = h_ref.shape[1] // 2
    dinv = lax.rsqrt(1.0 + a_ref[0, :, 0:1] + b_ref[0, :, 0:1])
    u = h_ref[...] * dinv
    o_ref[0] = u[:, :hw]
    o_ref[1] = u[:, hw:]


def _dg0(i):
    return (0, i, 0)


def _dg1(i):
    return (1, i, 0)


def _tc_scale(h, deg):
    n, m = h.shape
    hw = m // 2
    return pl.pallas_call(
        _scale_body,
        grid=(n // _RB,),
        in_specs=[
            pl.BlockSpec((_RB, m), lambda i: (i, 0)),
            pl.BlockSpec((1, _RB, 16), _dg0),
            pl.BlockSpec((1, _RB, 16), _dg1),
        ],
        out_specs=pl.BlockSpec((2, _RB, hw), lambda i: (0, i, 0)),
        out_shape=jax.ShapeDtypeStruct((2, n, hw), jnp.float32),
    )(h, deg, deg)


def _l2_body(pa0_ref, pb0_ref, pa1_ref, pb1_ref, ua_ref, ub_ref,
             a_ref, b_ref, b1a_ref, b1b_ref, w2a_ref, w2b_ref, o_ref):
    dinv = lax.rsqrt(1.0 + a_ref[0, :, 0:1] + b_ref[0, :, 0:1])
    t0 = dinv * (pa0_ref[0, 0] + pb0_ref[0, 0] + ua_ref[0]) + b1a_ref[0]
    t1 = dinv * (pa1_ref[0, 0] + pb1_ref[0, 0] + ub_ref[0]) + b1b_ref[0]
    t0 = jnp.maximum(t0, 0.0)
    t1 = jnp.maximum(t1, 0.0)
    o_ref[...] = dinv * (
        jnp.dot(t0, w2a_ref[...], preferred_element_type=jnp.float32)
        + jnp.dot(t1, w2b_ref[...], preferred_element_type=jnp.float32))


def _tc_layer2(p4, uh, deg, b1, w2):
    _, n, hw = uh.shape
    ncls = w2.shape[1]
    psp = lambda cc, hh: pl.BlockSpec((1, 1, _RB, hw),
                                      lambda i: (cc, hh, i, 0))
    return pl.pallas_call(
        _l2_body,
        grid=(n // _RB,),
        in_specs=[
            psp(0, 0), psp(1, 0), psp(0, 1), psp(1, 1),
            pl.BlockSpec((1, _RB, hw), lambda i: (0, i, 0)),
            pl.BlockSpec((1, _RB, hw), lambda i: (1, i, 0)),
            pl.BlockSpec((1, _RB, 16), _dg0),
            pl.BlockSpec((1, _RB, 16), _dg1),
            pl.BlockSpec((1, 1, hw), lambda i: (0, 0, 0)),
            pl.BlockSpec((1, 1, hw), lambda i: (1, 0, 0)),
            pl.BlockSpec((hw, ncls), lambda i: (0, 0)),
            pl.BlockSpec((hw, ncls), lambda i: (1, 0)),
        ],
        out_specs=pl.BlockSpec((_RB, ncls), lambda i: (i, 0)),
        out_shape=jax.ShapeDtypeStruct((n, ncls), jnp.float32),
    )(p4, p4, p4, p4, uh, uh, deg, deg, b1, b1, w2, w2)


def _fin_body(q0_ref, q1_ref, u2_ref, a_ref, b_ref, b2_ref, o_ref):
    dinv = lax.rsqrt(1.0 + a_ref[0, :, 0:1] + b_ref[0, :, 0:1])
    z = dinv * (q0_ref[0] + q1_ref[0] + u2_ref[...]) + b2_ref[...]
    z = z - jnp.max(z, axis=1, keepdims=True)
    e = jnp.exp(z)
    o_ref[...] = e / jnp.sum(e, axis=1, keepdims=True)


def _tc_final(q, u2, deg, b2):
    n, ncls = u2.shape
    return pl.pallas_call(
        _fin_body,
        grid=(n // _RB,),
        in_specs=[
            pl.BlockSpec((1, _RB, ncls), _dg0),
            pl.BlockSpec((1, _RB, ncls), _dg1),
            pl.BlockSpec((_RB, ncls), lambda i: (i, 0)),
            pl.BlockSpec((1, _RB, 16), _dg0),
            pl.BlockSpec((1, _RB, 16), _dg1),
            pl.BlockSpec((1, ncls), lambda i: (0, 0)),
        ],
        out_specs=pl.BlockSpec((_RB, ncls), lambda i: (i, 0)),
        out_shape=jax.ShapeDtypeStruct((n, ncls), jnp.float32),
    )(q, q, u2, deg, deg, b2)


def kernel(x, edge_index, W1, b1, W2, b2):
    n, _ = x.shape
    e = edge_index.shape[1]

    C = _cdiv(e, _NSLAB * _CH)
    epad = _NSLAB * C * _CH
    npad = (n // (16 * _CH) + 1) * (16 * _CH)

    pad = epad - e
    src = edge_index[0]
    dst = edge_index[1]
    src2 = jnp.concatenate(
        [src, jnp.zeros((pad,), jnp.int32)]).reshape(_NSLAB * C, _CH)
    dst2 = jnp.concatenate(
        [dst, jnp.full((pad,), n, jnp.int32)]).reshape(_NSLAB * C, _CH)

    tot = _NSLAB * C // 16

    deg = _sc_degree(dst2.reshape(_NSLAB, C, _CH), npad)

    h1 = _tc_matmul(x, W1)
    u1h = _tc_scale(h1, deg)

    p4 = _sc_agg128(u1h, src2, dst2, npad)
    u2 = _tc_layer2(p4, u1h, deg, b1.reshape(2, 1, -1), W2)

    q = _sc_agg_spmem(u2, src2, dst2, npad)
    return _tc_final(q, u2, deg, b2.reshape(1, -1))


# speedup vs baseline: 2.0405x; 1.0622x over previous
import functools

import jax
import jax.numpy as jnp
from jax import lax
from jax.experimental import pallas as pl
from jax.experimental.pallas import tpu as pltpu
from jax.experimental.pallas import tpu_sc as plsc

_CH = 128
_NSLAB = 32
_RB = 2000


def _cdiv(a, b):
    return (a + b - 1) // b


def _fill_const(ref, rows, d, val):
    vec = jnp.full((16,), val, jnp.float32)

    def row(i, carry):
        for jj in range(d // 16):
            ref[i, pl.ds(jj * 16, 16)] = vec
        return carry

    lax.fori_loop(0, rows, row, 0)


def _sc_degree(dst3, npad):
    nslab, C, ch = dst3.shape
    rpt = npad // 16
    mesh = plsc.VectorSubcoreMesh(core_axis_name="c", subcore_axis_name="s")

    def body(dst_hbm, out_hbm, didx, obuf, zbuf, acc):
        c = lax.axis_index("c")
        s = lax.axis_index("s")
        slab = c * 16 + s
        pltpu.sync_copy(dst_hbm.at[slab], didx)
        _fill_const(obuf, ch, 16, 1.0)
        _fill_const(zbuf, ch, 16, 0.0)
        for t in range(rpt // ch):
            pltpu.sync_copy(zbuf, acc.at[pl.ds(s * rpt + t * ch, ch)])
        plsc.subcore_barrier()

        def step(j, carry):
            pltpu.sync_copy(obuf, acc.at[didx.at[j]], add=True)
            return carry

        lax.fori_loop(0, C, step, 0)
        plsc.subcore_barrier()
        pltpu.sync_copy(acc.at[pl.ds(s * rpt, rpt)],
                        out_hbm.at[c, pl.ds(s * rpt, rpt)])

    f = pl.kernel(
        body,
        out_type=jax.ShapeDtypeStruct((2, npad, 16), jnp.float32),
        mesh=mesh,
        compiler_params=pltpu.CompilerParams(use_tc_tiling_on_sc=False),
        scratch_types=[
            pltpu.VMEM((C, ch), jnp.int32),
            pltpu.VMEM((ch, 16), jnp.float32),
            pltpu.VMEM((ch, 16), jnp.float32),
            pltpu.VMEM_SHARED((npad, 16), jnp.float32),
        ],
    )
    return f(dst3)


def _sc_agg128(uh, src2, dst2, npad):
    _, n, d = uh.shape
    tch, ch = src2.shape
    rpt = npad // 16
    nrt = n // 16
    cpt = tch // 16
    mesh = plsc.VectorSubcoreMesh(core_axis_name="c", subcore_axis_name="s")

    def body(u_hbm, src_hbm, dst_hbm, out_hbm, sidx3, didx3, rows, ushr,
             acc, isem, gsem, ssem):
        c = lax.axis_index("c")
        s = lax.axis_index("s")
        base = s * cpt
        C = cpt
        pltpu.sync_copy(u_hbm.at[c, pl.ds(s * nrt, nrt)],
                        ushr.at[pl.ds(s * nrt, nrt)])
        _fill_const(rows.at[0], ch, d, 0.0)
        for t in range(rpt // ch):
            pltpu.sync_copy(rows.at[0], acc.at[pl.ds(s * rpt + t * ch, ch)])
        plsc.subcore_barrier()

        def load_idx(j, slot, sem):
            pltpu.async_copy(src_hbm.at[base + j], sidx3.at[slot], sem)
            pltpu.async_copy(dst_hbm.at[base + j], didx3.at[slot], sem)

        def wait_idx(j, slot, sem):
            pltpu.make_async_copy(src_hbm.at[base + j], sidx3.at[slot],
                                  sem).wait()
            pltpu.make_async_copy(dst_hbm.at[base + j], didx3.at[slot],
                                  sem).wait()

        load_idx(0, 0, isem)
        wait_idx(0, 0, isem)
        pltpu.async_copy(ushr.at[sidx3.at[0]], rows.at[0], gsem)
        load_idx(1, 1, isem)

        def step(j, carry):
            b2 = j % 2
            b3 = j % 3
            pltpu.make_async_copy(ushr.at[sidx3.at[b3]], rows.at[b2],
                                  gsem).wait()

            @pl.when(j > 0)
            def _():
                pltpu.make_async_copy(rows.at[1 - b2],
                                      acc.at[didx3.at[(j + 2) % 3]],
                                      ssem).wait()

            pltpu.async_copy(rows.at[b2], acc.at[didx3.at[b3]], ssem,
                             add=True)

            @pl.when(j + 1 < C)
            def _():
                wait_idx(j + 1, (j + 1) % 3, isem)
                pltpu.async_copy(ushr.at[sidx3.at[(j + 1) % 3]],
                                 rows.at[1 - b2], gsem)

            @pl.when(j + 2 < C)
            def _():
                load_idx(j + 2, (j + 2) % 3, isem)

            return carry

        lax.fori_loop(0, C, step, 0)
        bl2 = (C - 1) % 2
        bl3 = (C - 1) % 3
        pltpu.make_async_copy(rows.at[bl2], acc.at[didx3.at[bl3]],
                              ssem).wait()
        plsc.subcore_barrier()
        pltpu.sync_copy(acc.at[pl.ds(s * rpt, rpt)],
                        out_hbm.at[c, pl.ds(s * rpt, rpt)])

    f = pl.kernel(
        body,
        out_type=jax.ShapeDtypeStruct((2, npad, d), jnp.float32),
        mesh=mesh,
        compiler_params=pltpu.CompilerParams(use_tc_tiling_on_sc=False),
        scratch_types=[
            pltpu.VMEM((3, ch), jnp.int32),
            pltpu.VMEM((3, ch), jnp.int32),
            pltpu.VMEM((2, ch, d), jnp.float32),
            pltpu.VMEM_SHARED((n, d), jnp.float32),
            pltpu.VMEM_SHARED((npad, d), jnp.float32),
            pltpu.SemaphoreType.DMA,
            pltpu.SemaphoreType.DMA,
            pltpu.SemaphoreType.DMA,
        ],
    )
    return f(uh, src2, dst2)


def _sc_agg_spmem(u, src2, dst2, npad):
    n, d = u.shape
    tch, ch = src2.shape
    rpt = npad // 16
    nrt = n // 16
    t_half = tch // 32
    mesh = plsc.VectorSubcoreMesh(core_axis_name="c", subcore_axis_name="s")

    def body(u_hbm, src_hbm, dst_hbm, out_hbm, sidx3, didx3, rows, ushr,
             acc, isem, gsem, ssem):
        c = lax.axis_index("c")
        s = lax.axis_index("s")
        base = (c * 16 + s) * t_half
        C = t_half
        pltpu.sync_copy(u_hbm.at[pl.ds(s * nrt, nrt)],
                        ushr.at[pl.ds(s * nrt, nrt)])
        _fill_const(rows.at[0], ch, d, 0.0)
        for t in range(rpt // ch):
            pltpu.sync_copy(rows.at[0], acc.at[pl.ds(s * rpt + t * ch, ch)])
        plsc.subcore_barrier()

        def load_idx(j, slot, sem):
            pltpu.async_copy(src_hbm.at[base + j], sidx3.at[slot], sem)
            pltpu.async_copy(dst_hbm.at[base + j], didx3.at[slot], sem)

        def wait_idx(j, slot, sem):
            pltpu.make_async_copy(src_hbm.at[base + j], sidx3.at[slot],
                                  sem).wait()
            pltpu.make_async_copy(dst_hbm.at[base + j], didx3.at[slot],
                                  sem).wait()

        load_idx(0, 0, isem)
        wait_idx(0, 0, isem)
        pltpu.async_copy(ushr.at[sidx3.at[0]], rows.at[0], gsem)
        load_idx(1, 1, isem)

        def step(j, carry):
            b2 = j % 2
            b3 = j % 3
            pltpu.make_async_copy(ushr.at[sidx3.at[b3]], rows.at[b2],
                                  gsem).wait()

            @pl.when(j > 0)
            def _():
                pltpu.make_async_copy(rows.at[1 - b2],
                                      acc.at[didx3.at[(j + 2) % 3]],
                                      ssem).wait()

            pltpu.async_copy(rows.at[b2], acc.at[didx3.at[b3]], ssem,
                             add=True)

            @pl.when(j + 1 < C)
            def _():
                wait_idx(j + 1, (j + 1) % 3, isem)
                pltpu.async_copy(ushr.at[sidx3.at[(j + 1) % 3]],
                                 rows.at[1 - b2], gsem)

            @pl.when(j + 2 < C)
            def _():
                load_idx(j + 2, (j + 2) % 3, isem)

            return carry

        lax.fori_loop(0, C, step, 0)
        bl2 = (C - 1) % 2
        bl3 = (C - 1) % 3
        pltpu.make_async_copy(rows.at[bl2], acc.at[didx3.at[bl3]],
                              ssem).wait()
        plsc.subcore_barrier()
        pltpu.sync_copy(acc.at[pl.ds(s * rpt, rpt)],
                        out_hbm.at[c, pl.ds(s * rpt, rpt)])

    f = pl.kernel(
        body,
        out_type=jax.ShapeDtypeStruct((2, npad, d), jnp.float32),
        mesh=mesh,
        compiler_params=pltpu.CompilerParams(use_tc_tiling_on_sc=False),
        scratch_types=[
            pltpu.VMEM((3, ch), jnp.int32),
            pltpu.VMEM((3, ch), jnp.int32),
            pltpu.VMEM((2, ch, d), jnp.float32),
            pltpu.VMEM_SHARED((n, d), jnp.float32),
            pltpu.VMEM_SHARED((npad, d), jnp.float32),
            pltpu.SemaphoreType.DMA,
            pltpu.SemaphoreType.DMA,
            pltpu.SemaphoreType.DMA,
        ],
    )
    return f(u, src2, dst2)


def _sc_agg(u, src2, dst2, npad, t0, t1):
    n, d = u.shape
    tch, ch = src2.shape
    assert 16 * (t0 + t1) == tch
    rpt = npad // 16
    mesh = plsc.VectorSubcoreMesh(core_axis_name="c", subcore_axis_name="s")

    def body(u_hbm, src_hbm, dst_hbm, out_hbm, sidx3, didx3, rows, acc,
             isem, gsem, ssem):
        c = lax.axis_index("c")
        s = lax.axis_index("s")
        C = jnp.where(c == 0, t0, t1)
        base = jnp.where(c == 0, s * t0, 16 * t0 + s * t1)
        _fill_const(rows.at[0], ch, d, 0.0)
        for t in range(rpt // ch):
            pltpu.sync_copy(rows.at[0], acc.at[pl.ds(s * rpt + t * ch, ch)])
        plsc.subcore_barrier()

        def load_idx(j, slot, sem):
            pltpu.async_copy(src_hbm.at[base + j], sidx3.at[slot], sem)
            pltpu.async_copy(dst_hbm.at[base + j], didx3.at[slot], sem)

        def wait_idx(j, slot, sem):
            pltpu.make_async_copy(src_hbm.at[base + j], sidx3.at[slot],
                                  sem).wait()
            pltpu.make_async_copy(dst_hbm.at[base + j], didx3.at[slot],
                                  sem).wait()

        load_idx(0, 0, isem)
        wait_idx(0, 0, isem)
        pltpu.async_copy(u_hbm.at[sidx3.at[0]], rows.at[0], gsem)
        load_idx(1, 1, isem)

        def step(j, carry):
            b2 = j % 2
            b3 = j % 3
            pltpu.make_async_copy(u_hbm.at[sidx3.at[b3]], rows.at[b2],
                                  gsem).wait()

            @pl.when(j > 0)
            def _():
                pltpu.make_async_copy(rows.at[1 - b2],
                                      acc.at[didx3.at[(j + 2) % 3]],
                                      ssem).wait()

            pltpu.async_copy(rows.at[b2], acc.at[didx3.at[b3]], ssem,
                             add=True)

            @pl.when(j + 1 < C)
            def _():
                wait_idx(j + 1, (j + 1) % 3, isem)
                pltpu.async_copy(u_hbm.at[sidx3.at[(j + 1) % 3]],
                                 rows.at[1 - b2], gsem)

            @pl.when(j + 2 < C)
            def _():
                load_idx(j + 2, (j + 2) % 3, isem)

            return carry

        lax.fori_loop(0, C, step, 0)
        bl2 = (C - 1) % 2
        bl3 = (C - 1) % 3
        pltpu.make_async_copy(rows.at[bl2], acc.at[didx3.at[bl3]],
                              ssem).wait()
        plsc.subcore_barrier()
        pltpu.sync_copy(acc.at[pl.ds(s * rpt, rpt)],
                        out_hbm.at[c, pl.ds(s * rpt, rpt)])

    f = pl.kernel(
        body,
        out_type=jax.ShapeDtypeStruct((2, npad, d), jnp.float32),
        mesh=mesh,
        compiler_params=pltpu.CompilerParams(use_tc_tiling_on_sc=False),
        scratch_types=[
            pltpu.VMEM((3, ch), jnp.int32),
            pltpu.VMEM((3, ch), jnp.int32),
            pltpu.VMEM((2, ch, d), jnp.float32),
            pltpu.VMEM_SHARED((npad, d), jnp.float32),
            pltpu.SemaphoreType.DMA,
            pltpu.SemaphoreType.DMA,
            pltpu.SemaphoreType.DMA,
        ],
    )
    return f(u, src2, dst2)


def _mm_body(x_ref, w_ref, o_ref):
    o_ref[...] = jnp.dot(x_ref[...], w_ref[...],
                         preferred_element_type=jnp.float32)


def _tc_matmul(x, w):
    n, k = x.shape
    m = w.shape[1]
    return pl.pallas_call(
        _mm_body,
        grid=(n // _RB,),
        in_specs=[
            pl.BlockSpec((_RB, k), lambda i: (i, 0)),
            pl.BlockSpec((k, m), lambda i: (0, 0)),
        ],
        out_specs=pl.BlockSpec((_RB, m), lambda i: (i, 0)),
        out_shape=jax.ShapeDtypeStruct((n, m), jnp.float32),
    )(x, w)


def _scale_body(h_ref, a_ref, b_ref, o_ref):
    hw = h_ref.shape[1] // 2
    dinv = lax.rsqrt(1.0 + a_ref[0, :, 0:1] + b_ref[0, :, 0:1])
    u = h_ref[...] * dinv
    o_ref[0] = u[:, :hw]
    o_ref[1] = u[:, hw:]


def _dg0(i):
    return (0, i, 0)


def _dg1(i):
    return (1, i, 0)


def _tc_scale(h, deg):
    n, m = h.shape
    hw = m // 2
    return pl.pallas_call(
        _scale_body,
        grid=(n // _RB,),
        in_specs=[
            pl.BlockSpec((_RB, m), lambda i: (i, 0)),
            pl.BlockSpec((1, _RB, 16), _dg0),
            pl.BlockSpec((1, _RB, 16), _dg1),
        ],
        out_specs=pl.BlockSpec((2, _RB, hw), lambda i: (0, i, 0)),
        out_shape=jax.ShapeDtypeStruct((2, n, hw), jnp.float32),
    )(h, deg, deg)


def _l2_body(p0_ref, p1_ref, ua_ref, ub_ref,
             a_ref, b_ref, b1a_ref, b1b_ref, w2a_ref, w2b_ref, o_ref):
    dinv = lax.rsqrt(1.0 + a_ref[0, :, 0:1] + b_ref[0, :, 0:1])
    t0 = dinv * (p0_ref[0] + ua_ref[0]) + b1a_ref[0]
    t1 = dinv * (p1_ref[0] + ub_ref[0]) + b1b_ref[0]
    t0 = jnp.maximum(t0, 0.0)
    t1 = jnp.maximum(t1, 0.0)
    o_ref[...] = dinv * (
        jnp.dot(t0, w2a_ref[...], preferred_element_type=jnp.float32)
        + jnp.dot(t1, w2b_ref[...], preferred_element_type=jnp.float32))


def _tc_layer2(p2, uh, deg, b1, w2):
    _, n, hw = uh.shape
    ncls = w2.shape[1]
    return pl.pallas_call(
        _l2_body,
        grid=(n // _RB,),
        in_specs=[
            pl.BlockSpec((1, _RB, hw), _dg0),
            pl.BlockSpec((1, _RB, hw), _dg1),
            pl.BlockSpec((1, _RB, hw), lambda i: (0, i, 0)),
            pl.BlockSpec((1, _RB, hw), lambda i: (1, i, 0)),
            pl.BlockSpec((1, _RB, 16), _dg0),
            pl.BlockSpec((1, _RB, 16), _dg1),
            pl.BlockSpec((1, 1, hw), lambda i: (0, 0, 0)),
            pl.BlockSpec((1, 1, hw), lambda i: (1, 0, 0)),
            pl.BlockSpec((hw, ncls), lambda i: (0, 0)),
            pl.BlockSpec((hw, ncls), lambda i: (1, 0)),
        ],
        out_specs=pl.BlockSpec((_RB, ncls), lambda i: (i, 0)),
        out_shape=jax.ShapeDtypeStruct((n, ncls), jnp.float32),
    )(p2, p2, uh, uh, deg, deg, b1, b1, w2, w2)


def _fin_body(q0_ref, q1_ref, u2_ref, a_ref, b_ref, b2_ref, o_ref):
    dinv = lax.rsqrt(1.0 + a_ref[0, :, 0:1] + b_ref[0, :, 0:1])
    z = dinv * (q0_ref[0] + q1_ref[0] + u2_ref[...]) + b2_ref[...]
    z = z - jnp.max(z, axis=1, keepdims=True)
    e = jnp.exp(z)
    o_ref[...] = e / jnp.sum(e, axis=1, keepdims=True)


def _tc_final(q, u2, deg, b2):
    n, ncls = u2.shape
    return pl.pallas_call(
        _fin_body,
        grid=(n // _RB,),
        in_specs=[
            pl.BlockSpec((1, _RB, ncls), _dg0),
            pl.BlockSpec((1, _RB, ncls), _dg1),
            pl.BlockSpec((_RB, ncls), lambda i: (i, 0)),
            pl.BlockSpec((1, _RB, 16), _dg0),
            pl.BlockSpec((1, _RB, 16), _dg1),
            pl.BlockSpec((1, ncls), lambda i: (0, 0)),
        ],
        out_specs=pl.BlockSpec((_RB, ncls), lambda i: (i, 0)),
        out_shape=jax.ShapeDtypeStruct((n, ncls), jnp.float32),
    )(q, q, u2, deg, deg, b2)


def kernel(x, edge_index, W1, b1, W2, b2):
    n, _ = x.shape
    e = edge_index.shape[1]

    C = _cdiv(e, _NSLAB * _CH)
    epad = _NSLAB * C * _CH
    npad = (n // (16 * _CH) + 1) * (16 * _CH)

    pad = epad - e
    src = edge_index[0]
    dst = edge_index[1]
    src2 = jnp.concatenate(
        [src, jnp.zeros((pad,), jnp.int32)]).reshape(_NSLAB * C, _CH)
    dst2 = jnp.concatenate(
        [dst, jnp.full((pad,), n, jnp.int32)]).reshape(_NSLAB * C, _CH)

    tot = _NSLAB * C // 16

    deg = _sc_degree(dst2.reshape(_NSLAB, C, _CH), npad)

    h1 = _tc_matmul(x, W1)
    u1h = _tc_scale(h1, deg)

    p4 = _sc_agg128(u1h, src2, dst2, npad)
    u2 = _tc_layer2(p4, u1h, deg, b1.reshape(2, 1, -1), W2)

    q = _sc_agg_spmem(u2, src2, dst2, npad)
    return _tc_final(q, u2, deg, b2.reshape(1, -1))
